# baseline (device time: 122629 ns/iter reference)
import jax
import jax.numpy as jnp
from jax import lax
from jax.experimental import pallas as pl
from jax.experimental.pallas import tpu as pltpu

N_DEV = 8


def kernel(x, Wg, Wu, Wd):
    m, d = x.shape
    _, h = Wg.shape

    def body(x_ref, wg_ref, wu_ref, wd_ref, out_ref, comm_ref, send_sems, recv_sems):
        my = lax.axis_index("i")
        left = lax.rem(my + (N_DEV - 1), N_DEV)
        right = lax.rem(my + 1, N_DEV)

        barrier_sem = pltpu.get_barrier_semaphore()
        for nbr in (left, right):
            pl.semaphore_signal(
                barrier_sem, inc=1,
                device_id=(nbr,), device_id_type=pl.DeviceIdType.MESH,
            )
        pl.semaphore_wait(barrier_sem, 2)

        xb = x_ref[...].astype(jnp.bfloat16)
        gate = jnp.dot(xb, wg_ref[...].astype(jnp.bfloat16),
                       preferred_element_type=jnp.float32)
        up = jnp.dot(xb, wu_ref[...].astype(jnp.bfloat16),
                     preferred_element_type=jnp.float32)
        act = (gate * (up * jax.nn.sigmoid(up))).astype(jnp.bfloat16)
        partial = jnp.dot(act, wd_ref[...].astype(jnp.bfloat16),
                          preferred_element_type=jnp.float32)

        out_ref[...] = partial
        comm_ref[0, :, :] = partial.astype(jnp.bfloat16)

        for k in range(N_DEV - 1):
            rdma = pltpu.make_async_remote_copy(
                src_ref=comm_ref.at[k],
                dst_ref=comm_ref.at[k + 1],
                send_sem=send_sems.at[k],
                recv_sem=recv_sems.at[k + 1],
                device_id=(right,),
                device_id_type=pl.DeviceIdType.MESH,
            )
            rdma.start()
            rdma.wait()
            out_ref[...] += comm_ref[k + 1, :, :].astype(jnp.float32)

    return pl.pallas_call(
        body,
        out_shape=jax.ShapeDtypeStruct((m, m), jnp.float32),
        in_specs=[
            pl.BlockSpec(memory_space=pltpu.VMEM),
            pl.BlockSpec(memory_space=pltpu.VMEM),
            pl.BlockSpec(memory_space=pltpu.VMEM),
            pl.BlockSpec(memory_space=pltpu.VMEM),
        ],
        out_specs=pl.BlockSpec(memory_space=pltpu.VMEM),
        scratch_shapes=[
            pltpu.VMEM((N_DEV, m, m), jnp.bfloat16),
            pltpu.SemaphoreType.DMA((N_DEV,)),
            pltpu.SemaphoreType.DMA((N_DEV,)),
        ],
        compiler_params=pltpu.CompilerParams(collective_id=0),
    )(x, Wg, Wu, Wd)


# device time: 52042 ns/iter; 2.3563x vs baseline; 2.3563x over previous
import jax
import jax.numpy as jnp
from jax import lax
from jax.experimental import pallas as pl
from jax.experimental.pallas import tpu as pltpu

N_DEV = 8

RS_MASKS = (3, 1, 4)
AG_MASKS = (4, 1, 3)


def kernel(x, Wg, Wu, Wd):
    m, d = x.shape
    _, h = Wg.shape
    half, qtr, eig = m // 2, m // 4, m // 8
    rs_sizes = (half, qtr, eig)
    ag_sizes = (eig, qtr, half)

    def body(x_ref, wg_ref, wu_ref, wd_ref, out_ref, acc_ref,
             stage0, stage1, stage2, recv0, recv1, recv2,
             send_sems, recv_sems):
        my = lax.axis_index("i")
        s0 = jnp.bitwise_and(jnp.right_shift(my, 1), 1)
        s1 = jnp.bitwise_and(my, 1)
        s2 = jnp.bitwise_and(jnp.right_shift(my, 2), 1)

        barrier_sem = pltpu.get_barrier_semaphore()
        for mask in RS_MASKS:
            pl.semaphore_signal(
                barrier_sem, inc=1,
                device_id=(jnp.bitwise_xor(my, mask),),
                device_id_type=pl.DeviceIdType.MESH,
            )
        pl.semaphore_wait(barrier_sem, 3)

        xb = x_ref[...].astype(jnp.bfloat16)
        gate = jnp.dot(xb, wg_ref[...].astype(jnp.bfloat16),
                       preferred_element_type=jnp.float32)
        up = jnp.dot(xb, wu_ref[...].astype(jnp.bfloat16),
                     preferred_element_type=jnp.float32)
        act = (gate * (up * jax.nn.sigmoid(up))).astype(jnp.bfloat16)
        acc_ref[...] = jnp.dot(act, wd_ref[...].astype(jnp.bfloat16),
                               preferred_element_type=jnp.float32)

        stages = (stage0, stage1, stage2)
        recvs = (recv0, recv1, recv2)
        sides = (s0, s1, s2)
        base = jnp.int32(0)
        for r, (mask, sz, side) in enumerate(zip(RS_MASKS, rs_sizes, sides)):
            partner = jnp.bitwise_xor(my, mask)
            keep_off = base + side * sz
            send_off = base + (1 - side) * sz
            stages[r][...] = acc_ref[pl.ds(send_off, sz), :].astype(jnp.bfloat16)
            rdma = pltpu.make_async_remote_copy(
                src_ref=stages[r],
                dst_ref=recvs[r],
                send_sem=send_sems.at[r],
                recv_sem=recv_sems.at[r],
                device_id=(partner,),
                device_id_type=pl.DeviceIdType.MESH,
            )
            rdma.start()
            rdma.wait()
            acc_ref[pl.ds(keep_off, sz), :] += recvs[r][...].astype(jnp.float32)
            base = keep_off

        my_off = base
        out_ref[pl.ds(my_off, eig), :] = (
            acc_ref[pl.ds(my_off, eig), :].astype(jnp.bfloat16))
        valid_offs = (my_off, s0 * half + s1 * qtr, s0 * half)
        for r, (mask, sz) in enumerate(zip(AG_MASKS, ag_sizes)):
            partner = jnp.bitwise_xor(my, mask)
            off = valid_offs[r]
            rdma = pltpu.make_async_remote_copy(
                src_ref=out_ref.at[pl.ds(off, sz), :],
                dst_ref=out_ref.at[pl.ds(off, sz), :],
                send_sem=send_sems.at[3 + r],
                recv_sem=recv_sems.at[3 + r],
                device_id=(partner,),
                device_id_type=pl.DeviceIdType.MESH,
            )
            rdma.start()
            rdma.wait()

    return pl.pallas_call(
        body,
        out_shape=jax.ShapeDtypeStruct((m, m), jnp.bfloat16),
        in_specs=[
            pl.BlockSpec(memory_space=pltpu.VMEM),
            pl.BlockSpec(memory_space=pltpu.VMEM),
            pl.BlockSpec(memory_space=pltpu.VMEM),
            pl.BlockSpec(memory_space=pltpu.VMEM),
        ],
        out_specs=pl.BlockSpec(memory_space=pltpu.VMEM),
        scratch_shapes=[
            pltpu.VMEM((m, m), jnp.float32),
            pltpu.VMEM((half, m), jnp.bfloat16),
            pltpu.VMEM((qtr, m), jnp.bfloat16),
            pltpu.VMEM((eig, m), jnp.bfloat16),
            pltpu.VMEM((half, m), jnp.bfloat16),
            pltpu.VMEM((qtr, m), jnp.bfloat16),
            pltpu.VMEM((eig, m), jnp.bfloat16),
            pltpu.SemaphoreType.DMA((6,)),
            pltpu.SemaphoreType.DMA((6,)),
        ],
        compiler_params=pltpu.CompilerParams(collective_id=0),
    )(x, Wg, Wu, Wd)
